# baseline (speedup 1.0000x reference)
import functools
import math

import jax
import jax.numpy as jnp
from jax import lax
from jax.experimental import pallas as pl
from jax.experimental.pallas import tpu as pltpu
from jax.experimental.pallas import tpu_sc as plsc

D, H, O, E, TOPK, T = 1024, 2048, 1024, 8, 2, 2048
CAP = T
BR = 256
RMAX = CAP // BR
NW = 32
TPW = T // NW
CHUNK = 32
WREP = 128
NBP = 24


def _gather16(v, idx):
    return jax.lax.gather(
        v, idx[:, None],
        jax.lax.GatherDimensionNumbers(
            offset_dims=(), collapsed_slice_dims=(0,), start_index_map=(0,)),
        slice_sizes=(1,),
        mode=jax.lax.GatherScatterMode.PROMISE_IN_BOUNDS)


def _router_body(x_ref, wg_ref, bg_ref, aux_ref, w0r_ref, w1r_ref,
                 d0_ref, d1_ref, exp_ref, rblk_ref, act_ref):
    x = x_ref[...]
    logits = jnp.dot(x, wg_ref[...], preferred_element_type=jnp.float32)
    logits = logits + bg_ref[...]
    m = jnp.max(logits, axis=-1, keepdims=True)
    p = jnp.exp(logits - m)
    probs = p / jnp.sum(p, axis=-1, keepdims=True)

    lane = jax.lax.broadcasted_iota(jnp.int32, (T, E), 1)
    m1 = jnp.max(probs, axis=-1, keepdims=True)
    i1 = jnp.argmax(probs, axis=-1)[:, None]
    probs2 = jnp.where(lane == i1, -1.0, probs)
    m2 = jnp.max(probs2, axis=-1, keepdims=True)
    i2 = jnp.argmax(probs2, axis=-1)[:, None]

    wsum = m1 + m2
    w0 = m1 / wsum
    w1 = m2 / wsum
    oh1 = (lane == i1).astype(jnp.float32)
    oh2 = (lane == i2).astype(jnp.float32)
    mask = oh1 + oh2

    c = mask
    s = 1
    while s < T:
        c = c + jnp.concatenate([jnp.zeros((s, E), jnp.float32), c[:-s, :]],
                                axis=0)
        s *= 2
    pos = c - mask
    pe0 = jnp.sum(pos * oh1, axis=1, keepdims=True)
    pe1 = jnp.sum(pos * oh2, axis=1, keepdims=True)
    d0_ref[...] = jnp.reshape(i1 * CAP + pe0.astype(jnp.int32), (T,))
    d1_ref[...] = jnp.reshape(i2 * CAP + pe1.astype(jnp.int32), (T,))
    w0r_ref[...] = jnp.broadcast_to(w0, (T, WREP))
    w1r_ref[...] = jnp.broadcast_to(w1, (T, WREP))
    counts = jnp.sum(mask, axis=0, keepdims=True).astype(jnp.int32)
    nb = (counts + BR - 1) // BR
    cnb = nb
    sh = 1
    while sh < E:
        cnb = cnb + jnp.concatenate(
            [jnp.zeros((1, sh), jnp.int32), cnb[:, :-sh]], axis=1)
        sh *= 2
    cnb_excl = cnb - nb
    gi = jax.lax.broadcasted_iota(jnp.int32, (NBP, E), 0)
    exp8 = jnp.sum((gi >= cnb[0][None, :]).astype(jnp.int32), axis=1,
                   keepdims=True)
    act = (exp8 < E).astype(jnp.int32)
    expc = jnp.minimum(exp8, E - 1)
    ohe = (jax.lax.broadcasted_iota(jnp.int32, (NBP, E), 1) == expc)
    cnbe = jnp.sum(cnb_excl[0][None, :] * ohe.astype(jnp.int32), axis=1,
                   keepdims=True)
    nbe = jnp.sum(nb[0][None, :] * ohe.astype(jnp.int32), axis=1,
                  keepdims=True)
    g_col = jax.lax.broadcasted_iota(jnp.int32, (NBP, 1), 0)
    rblk_raw = g_col - cnbe
    rblk = jnp.where(act == 1, rblk_raw, jnp.maximum(nbe - 1, 0))
    exp_ref[...] = jnp.reshape(expc, (1, NBP))
    rblk_ref[...] = jnp.reshape(rblk, (1, NBP))
    act_ref[...] = jnp.reshape(act, (1, NBP))

    imp = jnp.sum(probs, axis=0, keepdims=True)
    imp_mu = jnp.mean(imp)
    imp_std = jnp.sqrt(jnp.sum((imp - imp_mu) ** 2) / (E - 1))
    imp_loss = (imp_std / (imp_mu + 1e-6)) ** 2
    load = jnp.mean(mask, axis=0, keepdims=True)
    load_mu = jnp.mean(load)
    load_std = jnp.sqrt(jnp.sum((load - load_mu) ** 2) / (E - 1))
    load_loss = (load_std / (load_mu + 1e-6)) ** 2
    aux_ref[...] = (imp_loss + load_loss)[None, None]


def _run_router(x, Wg, bg):
    return pl.pallas_call(
        _router_body,
        out_shape=(
            jax.ShapeDtypeStruct((1, 1), jnp.float32),
            jax.ShapeDtypeStruct((T, WREP), jnp.float32),
            jax.ShapeDtypeStruct((T, WREP), jnp.float32),
            jax.ShapeDtypeStruct((T,), jnp.int32),
            jax.ShapeDtypeStruct((T,), jnp.int32),
            jax.ShapeDtypeStruct((1, NBP), jnp.int32),
            jax.ShapeDtypeStruct((1, NBP), jnp.int32),
            jax.ShapeDtypeStruct((1, NBP), jnp.int32),
        ),
    )(x, Wg, bg.reshape(1, E))


def _dispatch_body(x_hbm, d0_hbm, d1_hbm, w0r_hbm, w1r_hbm, xs_hbm, ws_hbm,
                   rows_v, idx_v, w_v, sem):
    wid = lax.axis_index("s") * 2 + lax.axis_index("c")
    base = wid * TPW
    for ci in range(TPW // CHUNK):
        off = base + ci * CHUNK
        pltpu.sync_copy(x_hbm.at[pl.ds(off, CHUNK)], rows_v)
        for d_hbm, wr_hbm in ((d0_hbm, w0r_hbm), (d1_hbm, w1r_hbm)):
            pltpu.sync_copy(d_hbm.at[pl.ds(off, CHUNK)], idx_v)
            pltpu.sync_copy(wr_hbm.at[pl.ds(off, CHUNK)], w_v)
            pltpu.async_copy(rows_v, xs_hbm.at[idx_v], sem).wait()
            pltpu.async_copy(w_v, ws_hbm.at[idx_v], sem).wait()


def _run_dispatch(x, d0, d1, w0r, w1r):
    mesh = plsc.VectorSubcoreMesh(core_axis_name="c", subcore_axis_name="s")
    f = pl.kernel(
        _dispatch_body,
        out_type=(
            jax.ShapeDtypeStruct((E * CAP, D), jnp.float32),
            jax.ShapeDtypeStruct((E * CAP, WREP), jnp.float32),
        ),
        mesh=mesh,
        scratch_types=[
            pltpu.VMEM((CHUNK, D), jnp.float32),
            pltpu.VMEM((CHUNK,), jnp.int32),
            pltpu.VMEM((CHUNK, WREP), jnp.float32),
            pltpu.SemaphoreType.DMA,
        ],
    )
    return f(x, d0, d1, w0r, w1r)


def _mm_body(exp_ref, rblk_ref, act_ref, xs_ref, ws_ref, w1_ref, b1_ref,
             lng_ref, lnb_ref, w2_ref, b2_ref, eo_ref, w1b_ref, w2b_ref):
    g = pl.program_id(0)

    @pl.when(rblk_ref[g] == 0)
    def _():
        w1b_ref[...] = w1_ref[0].astype(jnp.bfloat16)
        w2b_ref[...] = w2_ref[0].astype(jnp.bfloat16)

    @pl.when(act_ref[g] == 1)
    def _():
        h = jnp.dot(xs_ref[...].astype(jnp.bfloat16), w1b_ref[...],
                    preferred_element_type=jnp.float32) + b1_ref[0]
        mu = jnp.mean(h, axis=-1, keepdims=True)
        d = h - mu
        var = jnp.mean(d * d, axis=-1, keepdims=True)
        hn = d * jax.lax.rsqrt(var + 1e-5) * lng_ref[0] + lnb_ref[0]
        g = 0.5 * hn * (1.0 + jax.lax.erf(hn / math.sqrt(2.0)))
        eo = jnp.dot(g.astype(jnp.bfloat16), w2b_ref[...],
                     preferred_element_type=jnp.float32) + b2_ref[0]
        eo_ref[...] = eo * ws_ref[:, 0:1]


def _run_mm(exp_of, rblk_of, act_of, xs, ws, W1, b1, ln_g, ln_b, W2, b2):
    grid_spec = pltpu.PrefetchScalarGridSpec(
        num_scalar_prefetch=3,
        grid=(NBP,),
        in_specs=[
            pl.BlockSpec((BR, D), lambda g, ex, rb, ac: (
                ex[g] * RMAX + rb[g], 0)),
            pl.BlockSpec((BR, WREP), lambda g, ex, rb, ac: (
                ex[g] * RMAX + rb[g], 0)),
            pl.BlockSpec((1, D, H), lambda g, ex, rb, ac: (ex[g], 0, 0)),
            pl.BlockSpec((1, 1, H), lambda g, ex, rb, ac: (ex[g], 0, 0)),
            pl.BlockSpec((1, 1, H), lambda g, ex, rb, ac: (ex[g], 0, 0)),
            pl.BlockSpec((1, 1, H), lambda g, ex, rb, ac: (ex[g], 0, 0)),
            pl.BlockSpec((1, H, O), lambda g, ex, rb, ac: (ex[g], 0, 0)),
            pl.BlockSpec((1, 1, O), lambda g, ex, rb, ac: (ex[g], 0, 0)),
        ],
        out_specs=pl.BlockSpec((BR, O), lambda g, ex, rb, ac: (
            ex[g] * RMAX + rb[g], 0)),
        scratch_shapes=[
            pltpu.VMEM((D, H), jnp.bfloat16),
            pltpu.VMEM((H, O), jnp.bfloat16),
        ],
    )
    return pl.pallas_call(
        _mm_body,
        grid_spec=grid_spec,
        out_shape=jax.ShapeDtypeStruct((E * CAP, O), jnp.float32),
        compiler_params=pltpu.CompilerParams(
            dimension_semantics=("arbitrary",),
            vmem_limit_bytes=100 * 1024 * 1024,
        ),
    )(exp_of, rblk_of, act_of, xs, ws, W1, b1.reshape(E, 1, H),
      ln_g.reshape(E, 1, H), ln_b.reshape(E, 1, H), W2, b2.reshape(E, 1, O))


def _combine_body(eo_hbm, d0_hbm, d1_hbm, og_hbm, ob_hbm, out_hbm,
                  a_v, b_v, o_v, i0_v, i1_v, g_v, obv_v, sem):
    wid = lax.axis_index("s") * 2 + lax.axis_index("c")
    base = wid * TPW
    pltpu.sync_copy(og_hbm, g_v)
    pltpu.sync_copy(ob_hbm, obv_v)
    for ci in range(TPW // CHUNK):
        off = base + ci * CHUNK
        pltpu.sync_copy(d0_hbm.at[pl.ds(off, CHUNK)], i0_v)
        pltpu.sync_copy(d1_hbm.at[pl.ds(off, CHUNK)], i1_v)
        pltpu.async_copy(eo_hbm.at[i0_v], a_v, sem).wait()
        pltpu.async_copy(eo_hbm.at[i1_v], b_v, sem).wait()

        def row(i, carry):
            s = jnp.zeros((16,), jnp.float32)
            s2 = jnp.zeros((16,), jnp.float32)
            for l in range(O // 16):
                sl = pl.ds(l * 16, 16)
                v = a_v[i, sl] + b_v[i, sl]
                o_v[i, sl] = v
                s = s + v
                s2 = s2 + v * v
            for k in (1, 2, 4, 8):
                idx = jax.lax.iota(jnp.int32, 16) ^ k
                s = s + _gather16(s, idx)
                s2 = s2 + _gather16(s2, idx)
            mu = s * (1.0 / O)
            veps = s2 * (1.0 / O) - mu * mu + 1e-5
            y = 0.5 * (veps + 1.0)
            for _ in range(14):
                y = 0.5 * (y + veps / y)
            rr = 1.0 / y
            for l in range(O // 16):
                sl = pl.ds(l * 16, 16)
                o_v[i, sl] = (o_v[i, sl] - mu) * rr * g_v[sl] + obv_v[sl]
            return carry

        lax.fori_loop(0, CHUNK, row, 0)
        pltpu.sync_copy(o_v, out_hbm.at[pl.ds(off, CHUNK)])


def _run_combine(eo, d0, d1, out_g, out_b):
    mesh = plsc.VectorSubcoreMesh(core_axis_name="c", subcore_axis_name="s")
    f = pl.kernel(
        _combine_body,
        out_type=jax.ShapeDtypeStruct((T, O), jnp.float32),
        mesh=mesh,
        scratch_types=[
            pltpu.VMEM((CHUNK, O), jnp.float32),
            pltpu.VMEM((CHUNK, O), jnp.float32),
            pltpu.VMEM((CHUNK, O), jnp.float32),
            pltpu.VMEM((CHUNK,), jnp.int32),
            pltpu.VMEM((CHUNK,), jnp.int32),
            pltpu.VMEM((O,), jnp.float32),
            pltpu.VMEM((O,), jnp.float32),
            pltpu.SemaphoreType.DMA,
        ],
    )
    return f(eo, d0, d1, out_g, out_b)


def _ln_body(pre_ref, g_ref, b_ref, out_ref):
    y = pre_ref[...]
    mu = jnp.mean(y, axis=-1, keepdims=True)
    d = y - mu
    var = jnp.mean(d * d, axis=-1, keepdims=True)
    out_ref[...] = d * jax.lax.rsqrt(var + 1e-5) * g_ref[...] + b_ref[...]


def _run_ln(pre, out_g, out_b):
    return pl.pallas_call(
        _ln_body,
        grid=(T // 256,),
        in_specs=[
            pl.BlockSpec((256, O), lambda r: (r, 0)),
            pl.BlockSpec((1, O), lambda r: (0, 0)),
            pl.BlockSpec((1, O), lambda r: (0, 0)),
        ],
        out_specs=pl.BlockSpec((256, O), lambda r: (r, 0)),
        out_shape=jax.ShapeDtypeStruct((T, O), jnp.float32),
    )(pre, out_g.reshape(1, O), out_b.reshape(1, O))


@jax.jit
def kernel(x, Wg, bg, W1, b1, ln_g, ln_b, W2, b2, out_g, out_b):
    aux, w0r, w1r, d0, d1, exp_of, rblk_of, act_of = _run_router(x, Wg, bg)
    xs, ws = _run_dispatch(x, d0, d1, w0r, w1r)
    eo = _run_mm(exp_of.reshape(NBP), rblk_of.reshape(NBP),
                 act_of.reshape(NBP), xs, ws, W1, b1, ln_g, ln_b, W2, b2)
    out = _run_combine(eo, d0, d1, out_g, out_b)
    return (out, aux[0, 0])


# speedup vs baseline: 1.1417x; 1.1417x over previous
import math

import jax
import jax.numpy as jnp
from jax import lax
from jax.experimental import pallas as pl
from jax.experimental.pallas import tpu as pltpu
from jax.experimental.pallas import tpu_sc as plsc

D, H, O, E, TOPK, T = 1024, 2048, 1024, 8, 2, 2048
CAP = T
BR = 256
RMAX = CAP // BR
NW = 32
TPW = T // NW
CHUNK = 32
WREP = 128
NBP = 24


def _router_body(x_ref, wg_ref, bg_ref, aux_ref, w0r_ref, w1r_ref,
                 d0_ref, d1_ref, exp_ref, rblk_ref, act_ref):
    x = x_ref[...]
    logits = jnp.dot(x, wg_ref[...], preferred_element_type=jnp.float32)
    logits = logits + bg_ref[...]
    m = jnp.max(logits, axis=-1, keepdims=True)
    p = jnp.exp(logits - m)
    probs = p / jnp.sum(p, axis=-1, keepdims=True)

    lane = jax.lax.broadcasted_iota(jnp.int32, (T, E), 1)
    m1 = jnp.max(probs, axis=-1, keepdims=True)
    i1 = jnp.argmax(probs, axis=-1)[:, None]
    probs2 = jnp.where(lane == i1, -1.0, probs)
    m2 = jnp.max(probs2, axis=-1, keepdims=True)
    i2 = jnp.argmax(probs2, axis=-1)[:, None]

    wsum = m1 + m2
    w0 = m1 / wsum
    w1 = m2 / wsum
    oh1 = (lane == i1).astype(jnp.float32)
    oh2 = (lane == i2).astype(jnp.float32)
    mask = oh1 + oh2

    c = mask
    s = 1
    while s < T:
        c = c + jnp.concatenate([jnp.zeros((s, E), jnp.float32), c[:-s, :]],
                                axis=0)
        s *= 2
    pos = c - mask
    pe0 = jnp.sum(pos * oh1, axis=1, keepdims=True)
    pe1 = jnp.sum(pos * oh2, axis=1, keepdims=True)
    d0_ref[...] = jnp.reshape(i1 * CAP + pe0.astype(jnp.int32), (T,))
    d1_ref[...] = jnp.reshape(i2 * CAP + pe1.astype(jnp.int32), (T,))
    w0r_ref[...] = jnp.broadcast_to(w0, (T, WREP))
    w1r_ref[...] = jnp.broadcast_to(w1, (T, WREP))
    counts = jnp.sum(mask, axis=0, keepdims=True).astype(jnp.int32)
    nb = (counts + BR - 1) // BR
    cnb = nb
    sh = 1
    while sh < E:
        cnb = cnb + jnp.concatenate(
            [jnp.zeros((1, sh), jnp.int32), cnb[:, :-sh]], axis=1)
        sh *= 2
    cnb_excl = cnb - nb
    gi = jax.lax.broadcasted_iota(jnp.int32, (NBP, E), 0)
    exp8 = jnp.sum((gi >= cnb[0][None, :]).astype(jnp.int32), axis=1,
                   keepdims=True)
    act = (exp8 < E).astype(jnp.int32)
    expc = jnp.minimum(exp8, E - 1)
    ohe = (jax.lax.broadcasted_iota(jnp.int32, (NBP, E), 1) == expc)
    cnbe = jnp.sum(cnb_excl[0][None, :] * ohe.astype(jnp.int32), axis=1,
                   keepdims=True)
    nbe = jnp.sum(nb[0][None, :] * ohe.astype(jnp.int32), axis=1,
                  keepdims=True)
    g_col = jax.lax.broadcasted_iota(jnp.int32, (NBP, 1), 0)
    rblk_raw = g_col - cnbe
    rblk = jnp.where(act == 1, rblk_raw, jnp.maximum(nbe - 1, 0))
    exp_ref[...] = jnp.reshape(expc, (1, NBP))
    rblk_ref[...] = jnp.reshape(rblk, (1, NBP))
    act_ref[...] = jnp.reshape(act, (1, NBP))

    imp = jnp.sum(probs, axis=0, keepdims=True)
    imp_mu = jnp.mean(imp)
    imp_std = jnp.sqrt(jnp.sum((imp - imp_mu) ** 2) / (E - 1))
    imp_loss = (imp_std / (imp_mu + 1e-6)) ** 2
    load = jnp.mean(mask, axis=0, keepdims=True)
    load_mu = jnp.mean(load)
    load_std = jnp.sqrt(jnp.sum((load - load_mu) ** 2) / (E - 1))
    load_loss = (load_std / (load_mu + 1e-6)) ** 2
    aux_ref[...] = (imp_loss + load_loss)[None, None]


def _run_router(x, Wg, bg):
    return pl.pallas_call(
        _router_body,
        out_shape=(
            jax.ShapeDtypeStruct((1, 1), jnp.float32),
            jax.ShapeDtypeStruct((T, WREP), jnp.float32),
            jax.ShapeDtypeStruct((T, WREP), jnp.float32),
            jax.ShapeDtypeStruct((T,), jnp.int32),
            jax.ShapeDtypeStruct((T,), jnp.int32),
            jax.ShapeDtypeStruct((1, NBP), jnp.int32),
            jax.ShapeDtypeStruct((1, NBP), jnp.int32),
            jax.ShapeDtypeStruct((1, NBP), jnp.int32),
        ),
    )(x, Wg, bg.reshape(1, E))


def _dispatch_body(x_hbm, d0_hbm, d1_hbm, w0r_hbm, w1r_hbm, xs_hbm, ws_hbm,
                   rows_v, idx_v, w_v, sem):
    wid = lax.axis_index("s") * 2 + lax.axis_index("c")
    base = wid * TPW
    for ci in range(TPW // CHUNK):
        off = base + ci * CHUNK
        pltpu.sync_copy(x_hbm.at[pl.ds(off, CHUNK)], rows_v)
        for d_hbm, wr_hbm in ((d0_hbm, w0r_hbm), (d1_hbm, w1r_hbm)):
            pltpu.sync_copy(d_hbm.at[pl.ds(off, CHUNK)], idx_v)
            pltpu.sync_copy(wr_hbm.at[pl.ds(off, CHUNK)], w_v)
            pltpu.async_copy(rows_v, xs_hbm.at[idx_v], sem).wait()
            pltpu.async_copy(w_v, ws_hbm.at[idx_v], sem).wait()


def _run_dispatch(x, d0, d1, w0r, w1r):
    mesh = plsc.VectorSubcoreMesh(core_axis_name="c", subcore_axis_name="s")
    f = pl.kernel(
        _dispatch_body,
        out_type=(
            jax.ShapeDtypeStruct((E * CAP, D), jnp.float32),
            jax.ShapeDtypeStruct((E * CAP, WREP), jnp.float32),
        ),
        mesh=mesh,
        scratch_types=[
            pltpu.VMEM((CHUNK, D), jnp.float32),
            pltpu.VMEM((CHUNK,), jnp.int32),
            pltpu.VMEM((CHUNK, WREP), jnp.float32),
            pltpu.SemaphoreType.DMA,
        ],
    )
    return f(x, d0, d1, w0r, w1r)


def _mm_body(exp_ref, rblk_ref, act_ref, xs_ref, ws_ref, w1_ref, b1_ref,
             lng_ref, lnb_ref, w2_ref, b2_ref, eo_ref, w1b_ref, w2b_ref):
    g = pl.program_id(0)

    @pl.when(rblk_ref[g] == 0)
    def _():
        w1b_ref[...] = w1_ref[0].astype(jnp.bfloat16)
        w2b_ref[...] = w2_ref[0].astype(jnp.bfloat16)

    @pl.when(act_ref[g] == 1)
    def _():
        h = jnp.dot(xs_ref[...].astype(jnp.bfloat16), w1b_ref[...],
                    preferred_element_type=jnp.float32) + b1_ref[0]
        mu = jnp.mean(h, axis=-1, keepdims=True)
        d = h - mu
        var = jnp.mean(d * d, axis=-1, keepdims=True)
        hn = d * jax.lax.rsqrt(var + 1e-5) * lng_ref[0] + lnb_ref[0]
        g = 0.5 * hn * (1.0 + jax.lax.erf(hn / math.sqrt(2.0)))
        eo = jnp.dot(g.astype(jnp.bfloat16), w2b_ref[...],
                     preferred_element_type=jnp.float32) + b2_ref[0]
        eo_ref[...] = eo * ws_ref[:, 0:1]


def _run_mm(exp_of, rblk_of, act_of, xs, ws, W1, b1, ln_g, ln_b, W2, b2):
    grid_spec = pltpu.PrefetchScalarGridSpec(
        num_scalar_prefetch=3,
        grid=(NBP,),
        in_specs=[
            pl.BlockSpec((BR, D), lambda g, ex, rb, ac: (
                ex[g] * RMAX + rb[g], 0)),
            pl.BlockSpec((BR, WREP), lambda g, ex, rb, ac: (
                ex[g] * RMAX + rb[g], 0)),
            pl.BlockSpec((1, D, H), lambda g, ex, rb, ac: (ex[g], 0, 0)),
            pl.BlockSpec((1, 1, H), lambda g, ex, rb, ac: (ex[g], 0, 0)),
            pl.BlockSpec((1, 1, H), lambda g, ex, rb, ac: (ex[g], 0, 0)),
            pl.BlockSpec((1, 1, H), lambda g, ex, rb, ac: (ex[g], 0, 0)),
            pl.BlockSpec((1, H, O), lambda g, ex, rb, ac: (ex[g], 0, 0)),
            pl.BlockSpec((1, 1, O), lambda g, ex, rb, ac: (ex[g], 0, 0)),
        ],
        out_specs=pl.BlockSpec((BR, O), lambda g, ex, rb, ac: (
            ex[g] * RMAX + rb[g], 0)),
        scratch_shapes=[
            pltpu.VMEM((D, H), jnp.bfloat16),
            pltpu.VMEM((H, O), jnp.bfloat16),
        ],
    )
    return pl.pallas_call(
        _mm_body,
        grid_spec=grid_spec,
        out_shape=jax.ShapeDtypeStruct((E * CAP, O), jnp.float32),
        compiler_params=pltpu.CompilerParams(
            dimension_semantics=("arbitrary",),
            vmem_limit_bytes=100 * 1024 * 1024,
        ),
    )(exp_of, rblk_of, act_of, xs, ws, W1, b1.reshape(E, 1, H),
      ln_g.reshape(E, 1, H), ln_b.reshape(E, 1, H), W2, b2.reshape(E, 1, O))


def _combine_body(eo_hbm, d0_hbm, d1_hbm, pre_hbm,
                  a_v, b_v, o_v, i0_v, i1_v, sem):
    wid = lax.axis_index("s") * 2 + lax.axis_index("c")
    base = wid * TPW
    for ci in range(TPW // CHUNK):
        off = base + ci * CHUNK
        pltpu.sync_copy(d0_hbm.at[pl.ds(off, CHUNK)], i0_v)
        pltpu.sync_copy(d1_hbm.at[pl.ds(off, CHUNK)], i1_v)
        pltpu.async_copy(eo_hbm.at[i0_v], a_v, sem).wait()
        pltpu.async_copy(eo_hbm.at[i1_v], b_v, sem).wait()

        def row(i, carry):
            for l in range(O // 16):
                sl = pl.ds(l * 16, 16)
                o_v[i, sl] = a_v[i, sl] + b_v[i, sl]
            return carry

        lax.fori_loop(0, CHUNK, row, 0)
        pltpu.sync_copy(o_v, pre_hbm.at[pl.ds(off, CHUNK)])


def _run_combine(eo, d0, d1):
    mesh = plsc.VectorSubcoreMesh(core_axis_name="c", subcore_axis_name="s")
    f = pl.kernel(
        _combine_body,
        out_type=jax.ShapeDtypeStruct((T, O), jnp.float32),
        mesh=mesh,
        scratch_types=[
            pltpu.VMEM((CHUNK, O), jnp.float32),
            pltpu.VMEM((CHUNK, O), jnp.float32),
            pltpu.VMEM((CHUNK, O), jnp.float32),
            pltpu.VMEM((CHUNK,), jnp.int32),
            pltpu.VMEM((CHUNK,), jnp.int32),
            pltpu.SemaphoreType.DMA,
        ],
    )
    return f(eo, d0, d1)


def _ln_body(pre_ref, g_ref, b_ref, out_ref):
    y = pre_ref[...]
    mu = jnp.mean(y, axis=-1, keepdims=True)
    d = y - mu
    var = jnp.mean(d * d, axis=-1, keepdims=True)
    out_ref[...] = d * jax.lax.rsqrt(var + 1e-5) * g_ref[...] + b_ref[...]


def _run_ln(pre, out_g, out_b):
    return pl.pallas_call(
        _ln_body,
        grid=(T // 256,),
        in_specs=[
            pl.BlockSpec((256, O), lambda r: (r, 0)),
            pl.BlockSpec((1, O), lambda r: (0, 0)),
            pl.BlockSpec((1, O), lambda r: (0, 0)),
        ],
        out_specs=pl.BlockSpec((256, O), lambda r: (r, 0)),
        out_shape=jax.ShapeDtypeStruct((T, O), jnp.float32),
    )(pre, out_g.reshape(1, O), out_b.reshape(1, O))


@jax.jit
def kernel(x, Wg, bg, W1, b1, ln_g, ln_b, W2, b2, out_g, out_b):
    aux, w0r, w1r, d0, d1, exp_of, rblk_of, act_of = _run_router(x, Wg, bg)
    xs, ws = _run_dispatch(x, d0, d1, w0r, w1r)
    eo = _run_mm(exp_of.reshape(NBP), rblk_of.reshape(NBP),
                 act_of.reshape(NBP), xs, ws, W1, b1, ln_g, ln_b, W2, b2)
    pre = _run_combine(eo, d0, d1)
    out = _run_ln(pre, out_g, out_b)
    return (out, aux[0, 0])
